# Initial kernel scaffold; baseline (speedup 1.0000x reference)
import jax
import jax.numpy as jnp
from jax.experimental import pallas as pl


def kernel(x, edge_index, edge_attr, batch_idx, enc_W, enc_b, f_We1, f_Wm1, f_We2, f_Wm2, c_W, c_node, c_edge, t_We1, t_Wm1, t_We2, t_Wm2, p_W1, p_b1, p_W2, p_b2):
    raise NotImplementedError("write your pallas kernel here")


# speedup vs baseline: 1.1960x; 1.1960x over previous
import functools

import jax
import jax.numpy as jnp
from jax import lax
from jax.experimental import pallas as pl
from jax.experimental.pallas import tpu as pltpu
from jax.experimental.pallas import tpu_sc as plsc

N0 = 10000
E0 = 320000
G0 = 128
D0 = 300
NP = 10240
DP = 320
H = 160
NCORE = 2
NT = 16
EPT = E0 // NT
BE = 80
NBLK = EPT // BE
RPT = NP // NT


def _enc_body(x_ref, w_ref, b_ref, o_ref):
    o = jnp.dot(x_ref[...], w_ref[0], preferred_element_type=jnp.float32)
    o_ref[0] = jnp.maximum(o + b_ref[0], 0.0)


def _encode(xp, Wsp, bsp):
    BN = 512
    return pl.pallas_call(
        _enc_body,
        grid=(NP // BN, NCORE),
        in_specs=[
            pl.BlockSpec((BN, 128), lambda i, c: (i, 0)),
            pl.BlockSpec((1, 128, H), lambda i, c: (c, 0, 0)),
            pl.BlockSpec((1, 1, H), lambda i, c: (c, 0, 0)),
        ],
        out_specs=pl.BlockSpec((1, BN, H), lambda i, c: (c, i, 0)),
        out_shape=jax.ShapeDtypeStruct((NCORE, NP, H), jnp.float32),
    )(xp, Wsp, bsp)


def _emm_body(a_ref, w_ref, o_ref):
    o_ref[0] = jnp.dot(a_ref[...], w_ref[0],
                       preferred_element_type=jnp.float32)


def _edge_mm(ea, Wsp):
    BEB = 1600
    return pl.pallas_call(
        _emm_body,
        grid=(E0 // BEB, NCORE),
        in_specs=[
            pl.BlockSpec((BEB, 16), lambda i, c: (i, 0)),
            pl.BlockSpec((1, 16, H), lambda i, c: (c, 0, 0)),
        ],
        out_specs=pl.BlockSpec((1, BEB, H), lambda i, c: (c, i, 0)),
        out_shape=jax.ShapeDtypeStruct((NCORE, E0, H), jnp.float32),
    )(ea, Wsp)


def _nmm_body(h_ref, g_ref, w_ref, o_ref):
    s0 = h_ref[0] + g_ref[0]
    s1 = h_ref[1] + g_ref[1]
    w = w_ref[0]
    o = (jnp.dot(s0, w[:H], preferred_element_type=jnp.float32)
         + jnp.dot(s1, w[H:], preferred_element_type=jnp.float32))
    o_ref[0] = jnp.maximum(o, 0.0)


def _nmm_scaled_body(h_ref, g_ref, w_ref, n_ref, o_ref):
    sc = n_ref[...]
    s0 = h_ref[0] * sc + g_ref[0]
    s1 = h_ref[1] * sc + g_ref[1]
    w = w_ref[0]
    o = (jnp.dot(s0, w[:H], preferred_element_type=jnp.float32)
         + jnp.dot(s1, w[H:], preferred_element_type=jnp.float32))
    o_ref[0] = jnp.maximum(o, 0.0)


def _node_mm(h, g, Wsp, ncau=None):
    BN = 512
    in_specs = [
        pl.BlockSpec((NCORE, BN, H), lambda i, c: (0, i, 0)),
        pl.BlockSpec((NCORE, BN, H), lambda i, c: (0, i, 0)),
        pl.BlockSpec((1, DP, H), lambda i, c: (c, 0, 0)),
    ]
    args = [h, g, Wsp]
    body = _nmm_body
    if ncau is not None:
        in_specs.append(pl.BlockSpec((BN, 1), lambda i, c: (i, 0)))
        args.append(ncau)
        body = _nmm_scaled_body
    return pl.pallas_call(
        body,
        grid=(NP // BN, NCORE),
        in_specs=in_specs,
        out_specs=pl.BlockSpec((1, BN, H), lambda i, c: (c, i, 0)),
        out_shape=jax.ShapeDtypeStruct((NCORE, NP, H), jnp.float32),
    )(*args)


def _caus_body(g_ref, cw_ref, w3_ref, o_ref):
    a0 = g_ref[0]
    a1 = g_ref[1]
    cw = cw_ref[...]
    hc = jnp.maximum(
        jnp.dot(a0, cw[:H], preferred_element_type=jnp.float32)
        + jnp.dot(a1, cw[H:], preferred_element_type=jnp.float32), 0.0)
    p = jnp.dot(hc, w3_ref[...], preferred_element_type=jnp.float32)
    colid = lax.broadcasted_iota(jnp.int32, p.shape, 1)
    sig = 1.0 / (1.0 + jnp.exp(-p))
    o_ref[...] = jnp.where(colid == 0, sig, p)


def _causal(aggc, cWp, w3):
    BN = 512
    return pl.pallas_call(
        _caus_body,
        grid=(NP // BN,),
        in_specs=[
            pl.BlockSpec((NCORE, BN, H), lambda i: (0, i, 0)),
            pl.BlockSpec((DP, DP), lambda i: (0, 0)),
            pl.BlockSpec((DP, 8), lambda i: (0, 0)),
        ],
        out_specs=pl.BlockSpec((BN, 8), lambda i: (i, 0)),
        out_shape=jax.ShapeDtypeStruct((NP, 8), jnp.float32),
    )(aggc, cWp, w3)


def _final_body(h_ref, bi_ref, w1_ref, b1_ref, w2_ref, b2_ref, o_ref):
    bi = bi_ref[...]
    gids = lax.broadcasted_iota(jnp.int32, (G0, NP), 0)
    oh = (gids == bi).astype(jnp.float32)
    cnt = jnp.sum(oh, axis=1, keepdims=True)
    den = jnp.maximum(cnt, 1.0)
    hg0 = jnp.dot(oh, h_ref[0], preferred_element_type=jnp.float32) / den
    hg1 = jnp.dot(oh, h_ref[1], preferred_element_type=jnp.float32) / den
    w1 = w1_ref[...]
    z = (jnp.dot(hg0, w1[:H], preferred_element_type=jnp.float32)
         + jnp.dot(hg1, w1[H:], preferred_element_type=jnp.float32)
         + b1_ref[...])
    mu = jnp.mean(z, axis=0, keepdims=True)
    var = jnp.mean((z - mu) ** 2, axis=0, keepdims=True)
    z = jnp.maximum((z - mu) / jnp.sqrt(var + 1e-5), 0.0)
    o_ref[...] = (jnp.dot(z, w2_ref[...], preferred_element_type=jnp.float32)
                  + b2_ref[...])


def _final(h, bip, pW1p, b1r, pW2, b2r):
    return pl.pallas_call(
        _final_body,
        out_shape=jax.ShapeDtypeStruct((G0, 2), jnp.float32),
    )(h, bip, pW1p, b1r, pW2, b2r)


def _make_sc(mode):
    have_e = mode != "copy"
    have_c = mode == "cau"

    scratch = [
        pltpu.VMEM((BE,), jnp.int32),
        pltpu.VMEM((BE,), jnp.int32),
        pltpu.VMEM((BE,), jnp.int32),
        pltpu.VMEM((BE, H), jnp.float32),
        pltpu.VMEM_SHARED((NP, H), jnp.float32),
        pltpu.SemaphoreType.DMA,
    ]
    if have_e:
        scratch.append(pltpu.VMEM((BE, H), jnp.float32))
    if have_c:
        scratch.append(pltpu.VMEM((BE, 16), jnp.float32))
        scratch.append(pltpu.VMEM((BE, 16), jnp.float32))

    def body(*refs):
        pos = 0
        table = refs[pos]; pos += 1
        if have_e:
            ew = refs[pos]; pos += 1
        srcr = refs[pos]; dstr = refs[pos + 1]; pos += 2
        if have_c:
            ar = refs[pos]; br = refs[pos + 1]; pos += 2
        out = refs[pos]; pos += 1
        src_v, dst_v, gidx_v, gbuf, agg, sem = refs[pos:pos + 6]
        pos += 6
        if have_e:
            ebuf = refs[pos]; pos += 1
        if have_c:
            av, bv = refs[pos:pos + 2]; pos += 2

        c = lax.axis_index("c")
        s = lax.axis_index("s")

        zeros16 = jnp.zeros((16,), jnp.float32)

        def zrow(r, carry):
            for j in range(H // 16):
                gbuf[r, pl.ds(j * 16, 16)] = zeros16
            return carry

        lax.fori_loop(0, BE, zrow, 0)

        def zcp(k, carry):
            pltpu.sync_copy(gbuf, agg.at[pl.ds(s * RPT + k * BE, BE)])
            return carry

        lax.fori_loop(0, RPT // BE, zcp, 0)

        plsc.subcore_barrier()

        cbase = c * NP

        def eblk(ib, carry):
            e0 = s * EPT + ib * BE
            pltpu.sync_copy(srcr.at[pl.ds(e0, BE)], src_v)
            pltpu.sync_copy(dstr.at[pl.ds(e0, BE)], dst_v)
            for k in range(BE // 16):
                sl = pl.ds(k * 16, 16)
                gidx_v[sl] = src_v[sl] + cbase
            pltpu.async_copy(table.at[gidx_v], gbuf, sem).wait()
            if have_e:
                pltpu.sync_copy(ew.at[c, pl.ds(e0, BE)], ebuf)
            if have_c:
                pltpu.async_copy(ar.at[src_v], av, sem).wait()
                pltpu.async_copy(br.at[dst_v], bv, sem).wait()
            if have_e:
                def erow(b, carry2):
                    if have_c:
                        t = av[b, pl.ds(0, 16)] + bv[b, pl.ds(0, 16)]
                        cv = (1.0 / (1.0 + jnp.exp(-t)))[0]
                    for j in range(H // 16):
                        sl2 = pl.ds(j * 16, 16)
                        m = jnp.maximum(gbuf[b, sl2] + ebuf[b, sl2], 0.0)
                        if have_c:
                            m = m * cv
                        gbuf[b, sl2] = m
                    return carry2

                lax.fori_loop(0, BE, erow, 0)
            pltpu.sync_copy(gbuf, agg.at[dst_v], add=True)
            return carry

        lax.fori_loop(0, NBLK, eblk, 0)
        plsc.subcore_barrier()
        pltpu.sync_copy(agg.at[pl.ds(s * RPT, RPT)],
                        out.at[c, pl.ds(s * RPT, RPT)])

    mesh = plsc.VectorSubcoreMesh(core_axis_name="c", subcore_axis_name="s")
    return pl.kernel(
        body,
        out_type=jax.ShapeDtypeStruct((NCORE, NP, H), jnp.float32),
        mesh=mesh,
        scratch_types=scratch,
        compiler_params=pltpu.CompilerParams(use_tc_tiling_on_sc=False),
    )


_sc_cache = {}


def _get_sc(mode):
    if mode not in _sc_cache:
        _sc_cache[mode] = _make_sc(mode)
    return _sc_cache[mode]


def _seg_copy(*args):
    return _get_sc("copy")(*args)


def _seg_relu(*args):
    return _get_sc("relu")(*args)


def _seg_cau(*args):
    return _get_sc("cau")(*args)


def _pad_sq(w):
    return jnp.pad(w, ((0, DP - w.shape[0]), (0, DP - w.shape[1])))


def _pad_we(w):
    return jnp.pad(w, ((0, 0), (0, DP - w.shape[1])))


def _split_cols(w):
    return jnp.stack([w[:, :H], w[:, H:]], axis=0)


def kernel(x, edge_index, edge_attr, batch_idx, enc_W, enc_b,
           f_We1, f_Wm1, f_We2, f_Wm2, c_W, c_node, c_edge,
           t_We1, t_Wm1, t_We2, t_Wm2, p_W1, p_b1, p_W2, p_b2):
    src = edge_index[0]
    dst = edge_index[1]

    xp = jnp.pad(x, ((0, NP - N0), (0, 0)))
    encWs = _split_cols(jnp.pad(enc_W, ((0, 0), (0, DP - D0))))
    encbs = _split_cols(jnp.pad(enc_b, (0, DP - D0)).reshape(1, DP))
    w3 = jnp.concatenate([
        jnp.pad(c_node, ((0, DP - D0), (0, 0))),
        jnp.pad(c_edge[:D0], ((0, DP - D0), (0, 0))),
        jnp.pad(c_edge[D0:], ((0, DP - D0), (0, 0))),
        jnp.zeros((DP, 5), jnp.float32),
    ], axis=1)
    pW1p = jnp.pad(p_W1, ((0, DP - D0), (0, 0)))
    bip = jnp.pad(batch_idx, (0, NP - N0),
                  constant_values=G0).reshape(1, NP)

    h0 = _encode(xp, encWs, encbs)

    ew = _edge_mm(edge_attr, _split_cols(_pad_we(f_We1)))
    agg = _seg_relu(h0.reshape(NCORE * NP, H), ew, src, dst)
    h1 = _node_mm(h0, agg, _split_cols(_pad_sq(f_Wm1)))

    ew = _edge_mm(edge_attr, _split_cols(_pad_we(f_We2)))
    agg = _seg_relu(h1.reshape(NCORE * NP, H), ew, src, dst)
    xe = _node_mm(h1, agg, _split_cols(_pad_sq(f_Wm2)))

    aggc = _seg_copy(xe.reshape(NCORE * NP, H), src, dst)
    pout = _causal(aggc, _pad_sq(c_W), w3)
    ncau = pout[:, 0:1]
    a2 = jnp.pad(pout[:, 1:2], ((0, 0), (0, 15)))
    b2 = jnp.pad(pout[:, 2:3], ((0, 0), (0, 15)))

    ew = _edge_mm(edge_attr, _split_cols(_pad_we(t_We1)))
    agg = _seg_cau(xe.reshape(NCORE * NP, H), ew, src, dst, a2, b2)
    h3 = _node_mm(xe, agg, _split_cols(_pad_sq(t_Wm1)), ncau)

    ew = _edge_mm(edge_attr, _split_cols(_pad_we(t_We2)))
    agg = _seg_cau(h3.reshape(NCORE * NP, H), ew, src, dst, a2, b2)
    h4 = _node_mm(h3, agg, _split_cols(_pad_sq(t_Wm2)), ncau)

    return _final(h4, bip, pW1p, p_b1.reshape(1, -1), p_W2,
                  p_b2.reshape(1, -1))


# speedup vs baseline: 2.0239x; 1.6922x over previous
import functools

import jax
import jax.numpy as jnp
from jax import lax
from jax.experimental import pallas as pl
from jax.experimental.pallas import tpu as pltpu
from jax.experimental.pallas import tpu_sc as plsc

N0 = 10000
E0 = 320000
G0 = 128
D0 = 300
NP = 10112
DP = 320
H = 160
NCORE = 2
NT = 16
EPT = E0 // NT
BE = 80
NBLK = EPT // BE
RPT = NP // NT


def _enc_body(x_ref, w_ref, b_ref, o_ref):
    o = jnp.dot(x_ref[...], w_ref[0], preferred_element_type=jnp.float32)
    o_ref[0] = jnp.maximum(o + b_ref[0], 0.0)


def _encode(xp, Wsp, bsp):
    BN = 632
    return pl.pallas_call(
        _enc_body,
        grid=(NP // BN, NCORE),
        in_specs=[
            pl.BlockSpec((BN, 128), lambda i, c: (i, 0)),
            pl.BlockSpec((1, 128, H), lambda i, c: (c, 0, 0)),
            pl.BlockSpec((1, 1, H), lambda i, c: (c, 0, 0)),
        ],
        out_specs=pl.BlockSpec((1, BN, H), lambda i, c: (c, i, 0)),
        out_shape=jax.ShapeDtypeStruct((NCORE, NP, H), jnp.float32),
    )(xp, Wsp, bsp)


def _emm_body(a_ref, w_ref, o_ref):
    o_ref[0] = jnp.dot(a_ref[...], w_ref[0],
                       preferred_element_type=jnp.float32)


def _edge_mm(ea, Wsp):
    BEB = 1600
    return pl.pallas_call(
        _emm_body,
        grid=(E0 // BEB, NCORE),
        in_specs=[
            pl.BlockSpec((BEB, 16), lambda i, c: (i, 0)),
            pl.BlockSpec((1, 16, H), lambda i, c: (c, 0, 0)),
        ],
        out_specs=pl.BlockSpec((1, BEB, H), lambda i, c: (c, i, 0)),
        out_shape=jax.ShapeDtypeStruct((NCORE, E0, H), jnp.float32),
    )(ea, Wsp)


def _nmm_body(h_ref, g_ref, w_ref, o_ref):
    s0 = h_ref[0] + g_ref[0]
    s1 = h_ref[1] + g_ref[1]
    w = w_ref[0]
    o = (jnp.dot(s0, w[:H], preferred_element_type=jnp.float32)
         + jnp.dot(s1, w[H:], preferred_element_type=jnp.float32))
    o_ref[0] = jnp.maximum(o, 0.0)


def _nmm_scaled_body(h_ref, g_ref, w_ref, n_ref, o_ref):
    sc = n_ref[...]
    s0 = h_ref[0] * sc + g_ref[0]
    s1 = h_ref[1] * sc + g_ref[1]
    w = w_ref[0]
    o = (jnp.dot(s0, w[:H], preferred_element_type=jnp.float32)
         + jnp.dot(s1, w[H:], preferred_element_type=jnp.float32))
    o_ref[0] = jnp.maximum(o, 0.0)


def _node_mm(h, g, Wsp, ncau=None):
    BN = 632
    in_specs = [
        pl.BlockSpec((NCORE, BN, H), lambda i, c: (0, i, 0)),
        pl.BlockSpec((NCORE, BN, H), lambda i, c: (0, i, 0)),
        pl.BlockSpec((1, DP, H), lambda i, c: (c, 0, 0)),
    ]
    args = [h, g, Wsp]
    body = _nmm_body
    if ncau is not None:
        in_specs.append(pl.BlockSpec((BN, 1), lambda i, c: (i, 0)))
        args.append(ncau)
        body = _nmm_scaled_body
    return pl.pallas_call(
        body,
        grid=(NP // BN, NCORE),
        in_specs=in_specs,
        out_specs=pl.BlockSpec((1, BN, H), lambda i, c: (c, i, 0)),
        out_shape=jax.ShapeDtypeStruct((NCORE, NP, H), jnp.float32),
    )(*args)


def _caus_body(g_ref, cw_ref, w3_ref, o_ref):
    a0 = g_ref[0]
    a1 = g_ref[1]
    cw = cw_ref[...]
    hc = jnp.maximum(
        jnp.dot(a0, cw[:H], preferred_element_type=jnp.float32)
        + jnp.dot(a1, cw[H:], preferred_element_type=jnp.float32), 0.0)
    p = jnp.dot(hc, w3_ref[...], preferred_element_type=jnp.float32)
    colid = lax.broadcasted_iota(jnp.int32, p.shape, 1)
    sig = 1.0 / (1.0 + jnp.exp(-p))
    o_ref[...] = jnp.where(colid == 0, sig, p)


def _causal(aggc, cWp, w3):
    BN = 632
    return pl.pallas_call(
        _caus_body,
        grid=(NP // BN,),
        in_specs=[
            pl.BlockSpec((NCORE, BN, H), lambda i: (0, i, 0)),
            pl.BlockSpec((DP, DP), lambda i: (0, 0)),
            pl.BlockSpec((DP, 8), lambda i: (0, 0)),
        ],
        out_specs=pl.BlockSpec((BN, 8), lambda i: (i, 0)),
        out_shape=jax.ShapeDtypeStruct((NP, 8), jnp.float32),
    )(aggc, cWp, w3)


def _final_body(h_ref, bi_ref, w1_ref, b1_ref, w2_ref, b2_ref, o_ref):
    bi = bi_ref[...]
    gids = lax.broadcasted_iota(jnp.int32, (G0, NP), 0)
    oh = (gids == bi).astype(jnp.float32)
    cnt = jnp.sum(oh, axis=1, keepdims=True)
    den = jnp.maximum(cnt, 1.0)
    hg0 = jnp.dot(oh, h_ref[0], preferred_element_type=jnp.float32) / den
    hg1 = jnp.dot(oh, h_ref[1], preferred_element_type=jnp.float32) / den
    w1 = w1_ref[...]
    z = (jnp.dot(hg0, w1[:H], preferred_element_type=jnp.float32)
         + jnp.dot(hg1, w1[H:], preferred_element_type=jnp.float32)
         + b1_ref[...])
    mu = jnp.mean(z, axis=0, keepdims=True)
    var = jnp.mean((z - mu) ** 2, axis=0, keepdims=True)
    z = jnp.maximum((z - mu) / jnp.sqrt(var + 1e-5), 0.0)
    o_ref[...] = (jnp.dot(z, w2_ref[...], preferred_element_type=jnp.float32)
                  + b2_ref[...])


def _final(h, bip, pW1p, b1r, pW2, b2r):
    return pl.pallas_call(
        _final_body,
        out_shape=jax.ShapeDtypeStruct((G0, 2), jnp.float32),
    )(h, bip, pW1p, b1r, pW2, b2r)


def _make_sc(mode):
    have_e = mode != "copy"
    have_c = mode == "cau"

    per_set = [
        pltpu.VMEM((2, BE), jnp.int32),
        pltpu.VMEM((BE,), jnp.int32),
        pltpu.VMEM((BE,), jnp.int32),
        pltpu.VMEM((BE, H), jnp.float32),
    ]
    n_sem = 3 + (1 if have_e else 0) + (1 if have_c else 0)
    if have_c:
        per_set.append(pltpu.VMEM((BE, 16), jnp.float32))
    scratch = per_set + per_set + [
        pltpu.VMEM_SHARED((NP, H), jnp.float32),
    ] + [pltpu.SemaphoreType.DMA] * (2 * n_sem)

    def body(*refs):
        pos = 0
        table = refs[pos]; pos += 1
        if have_e:
            ew = refs[pos]; pos += 1
        eidx = refs[pos]; pos += 1
        if have_c:
            ar = refs[pos]; br = refs[pos + 1]; pos += 2
        out = refs[pos]; pos += 1

        nbuf = 4 + (1 if have_c else 0)
        sets = []
        for _ in range(2):
            d = {"idx2": refs[pos], "gidx": refs[pos + 1],
                 "dstv": refs[pos + 2], "gbuf": refs[pos + 3]}
            if have_c:
                d["av"] = refs[pos + 4]
            sets.append(d)
            pos += nbuf
        agg = refs[pos]; pos += 1
        for d in sets:
            d["si"] = refs[pos]; d["sg"] = refs[pos + 1]
            d["ss"] = refs[pos + 2]; pos += 3
            if have_e:
                d["se"] = refs[pos]; pos += 1
            if have_c:
                d["sab"] = refs[pos]; pos += 1
        A, B = sets

        c = lax.axis_index("c")
        s = lax.axis_index("s")
        cbase = c * NP

        zeros16 = jnp.zeros((16,), jnp.float32)

        def zrow(r, carry):
            for j in range(H // 16):
                A["gbuf"][r, pl.ds(j * 16, 16)] = zeros16
            return carry

        lax.fori_loop(0, BE, zrow, 0)

        nfull = RPT // BE

        def zcp(k, carry):
            pltpu.sync_copy(A["gbuf"], agg.at[pl.ds(s * RPT + k * BE, BE)])
            return carry

        lax.fori_loop(0, nfull, zcp, 0)
        rem = RPT - nfull * BE
        if rem:
            pltpu.sync_copy(A["gbuf"].at[pl.ds(0, rem)],
                            agg.at[pl.ds(s * RPT + nfull * BE, rem)])

        plsc.subcore_barrier()

        def p1(X, blk, first=False):
            e0 = s * EPT + blk * BE
            if not first:
                pltpu.make_async_copy(
                    X["gbuf"], agg.at[X["dstv"]], X["ss"]).wait()
            pltpu.async_copy(eidx.at[:, pl.ds(e0, BE)], X["idx2"], X["si"])
            if have_e:
                pltpu.async_copy(ew.at[c, pl.ds(e0, BE)], X["gbuf"], X["se"])

        def p2(X, blk):
            e0 = s * EPT + blk * BE
            pltpu.make_async_copy(
                eidx.at[:, pl.ds(e0, BE)], X["idx2"], X["si"]).wait()
            if have_c:
                pltpu.async_copy(ar.at[X["idx2"].at[0]], X["av"], X["sab"])
            for k in range(BE // 16):
                sl = pl.ds(k * 16, 16)
                X["gidx"][sl] = X["idx2"][0, sl] + cbase
                X["dstv"][sl] = X["idx2"][1, sl]
            if have_e:
                pltpu.make_async_copy(
                    ew.at[c, pl.ds(e0, BE)], X["gbuf"], X["se"]).wait()
                pltpu.async_copy(table.at[X["gidx"]], X["gbuf"], X["sg"],
                                 add=True)
            else:
                pltpu.async_copy(table.at[X["gidx"]], X["gbuf"], X["sg"])
            if have_c:
                pltpu.make_async_copy(
                    ar.at[X["idx2"].at[0]], X["av"], X["sab"]).wait()
                pltpu.async_copy(br.at[X["idx2"].at[1]], X["av"], X["sab"],
                                 add=True)

        def p3(X, blk):
            pltpu.make_async_copy(
                table.at[X["gidx"]], X["gbuf"], X["sg"]).wait()
            if have_c:
                pltpu.make_async_copy(
                    br.at[X["idx2"].at[1]], X["av"], X["sab"]).wait()
            if have_e:
                def erow(b, carry2):
                    if have_c:
                        t = X["av"][b, pl.ds(0, 16)]
                        cv = (1.0 / (1.0 + jnp.exp(-t)))[0]
                    for j in range(H // 16):
                        sl2 = pl.ds(j * 16, 16)
                        m = jnp.maximum(X["gbuf"][b, sl2], 0.0)
                        if have_c:
                            m = m * cv
                        X["gbuf"][b, sl2] = m
                    return carry2

                lax.fori_loop(0, BE, erow, 0)
            pltpu.async_copy(X["gbuf"], agg.at[X["dstv"]], X["ss"],
                             add=True)

        p1(A, 0, first=True)
        p2(A, 0)
        p1(B, 1, first=True)

        def step(i, carry):
            blk_a = 2 * i
            blk_b = 2 * i + 1
            nxt_a = jnp.minimum(blk_a + 2, NBLK - 1)
            nxt_b = jnp.minimum(blk_b + 2, NBLK - 1)
            p3(A, blk_a)
            p2(B, blk_b)
            p1(A, nxt_a)
            p3(B, blk_b)
            p2(A, nxt_a)
            p1(B, nxt_b)
            return carry

        lax.fori_loop(0, NBLK // 2, step, 0)

        e_last = s * EPT + (NBLK - 1) * BE
        pltpu.make_async_copy(table.at[A["gidx"]], A["gbuf"], A["sg"]).wait()
        if have_c:
            pltpu.make_async_copy(
                br.at[A["idx2"].at[1]], A["av"], A["sab"]).wait()
        pltpu.make_async_copy(
            eidx.at[:, pl.ds(e_last, BE)], B["idx2"], B["si"]).wait()
        if have_e:
            pltpu.make_async_copy(
                ew.at[c, pl.ds(e_last, BE)], B["gbuf"], B["se"]).wait()

        plsc.subcore_barrier()
        pltpu.sync_copy(agg.at[pl.ds(s * RPT, RPT)],
                        out.at[c, pl.ds(s * RPT, RPT)])

    mesh = plsc.VectorSubcoreMesh(core_axis_name="c", subcore_axis_name="s")
    return pl.kernel(
        body,
        out_type=jax.ShapeDtypeStruct((NCORE, NP, H), jnp.float32),
        mesh=mesh,
        scratch_types=scratch,
        compiler_params=pltpu.CompilerParams(use_tc_tiling_on_sc=False),
    )


_sc_cache = {}


def _get_sc(mode):
    if mode not in _sc_cache:
        _sc_cache[mode] = _make_sc(mode)
    return _sc_cache[mode]


def _seg_copy(*args):
    return _get_sc("copy")(*args)


def _seg_relu(*args):
    return _get_sc("relu")(*args)


def _seg_cau(*args):
    return _get_sc("cau")(*args)


def _pad_sq(w):
    return jnp.pad(w, ((0, DP - w.shape[0]), (0, DP - w.shape[1])))


def _pad_we(w):
    return jnp.pad(w, ((0, 0), (0, DP - w.shape[1])))


def _split_cols(w):
    return jnp.stack([w[:, :H], w[:, H:]], axis=0)


def kernel(x, edge_index, edge_attr, batch_idx, enc_W, enc_b,
           f_We1, f_Wm1, f_We2, f_Wm2, c_W, c_node, c_edge,
           t_We1, t_Wm1, t_We2, t_Wm2, p_W1, p_b1, p_W2, p_b2):
    xp = jnp.pad(x, ((0, NP - N0), (0, 0)))
    encWs = _split_cols(jnp.pad(enc_W, ((0, 0), (0, DP - D0))))
    encbs = _split_cols(jnp.pad(enc_b, (0, DP - D0)).reshape(1, DP))
    w3 = jnp.concatenate([
        jnp.pad(c_node, ((0, DP - D0), (0, 0))),
        jnp.pad(c_edge[:D0], ((0, DP - D0), (0, 0))),
        jnp.pad(c_edge[D0:], ((0, DP - D0), (0, 0))),
        jnp.zeros((DP, 5), jnp.float32),
    ], axis=1)
    pW1p = jnp.pad(p_W1, ((0, DP - D0), (0, 0)))
    bip = jnp.pad(batch_idx, (0, NP - N0),
                  constant_values=G0).reshape(1, NP)

    h0 = _encode(xp, encWs, encbs)

    ew = _edge_mm(edge_attr, _split_cols(_pad_we(f_We1)))
    agg = _seg_relu(h0.reshape(NCORE * NP, H), ew, edge_index)
    h1 = _node_mm(h0, agg, _split_cols(_pad_sq(f_Wm1)))

    ew = _edge_mm(edge_attr, _split_cols(_pad_we(f_We2)))
    agg = _seg_relu(h1.reshape(NCORE * NP, H), ew, edge_index)
    xe = _node_mm(h1, agg, _split_cols(_pad_sq(f_Wm2)))

    aggc = _seg_copy(xe.reshape(NCORE * NP, H), edge_index)
    pout = _causal(aggc, _pad_sq(c_W), w3)
    ncau = pout[:, 0:1]
    a2 = jnp.pad(pout[:, 1:2], ((0, 0), (0, 15)))
    b2 = jnp.pad(pout[:, 2:3], ((0, 0), (0, 15)))

    ew = _edge_mm(edge_attr, _split_cols(_pad_we(t_We1)))
    agg = _seg_cau(xe.reshape(NCORE * NP, H), ew, edge_index, a2, b2)
    h3 = _node_mm(xe, agg, _split_cols(_pad_sq(t_Wm1)), ncau)

    ew = _edge_mm(edge_attr, _split_cols(_pad_we(t_We2)))
    agg = _seg_cau(h3.reshape(NCORE * NP, H), ew, edge_index, a2, b2)
    h4 = _node_mm(h3, agg, _split_cols(_pad_sq(t_Wm2)), ncau)

    return _final(h4, bip, pW1p, p_b1.reshape(1, -1), p_W2,
                  p_b2.reshape(1, -1))
